# baseline (speedup 1.0000x reference)
import jax
import jax.numpy as jnp
from jax import lax
from jax.experimental import pallas as pl
from jax.experimental.pallas import tpu as pltpu
from jax.experimental.pallas import tpu_sc as plsc

N_NODES = 100000
N_EDGES = 6400000
NP = 100096
D = 8

GRP = 128
G_PER_CHUNK = 8
CHUNK = GRP * G_PER_CHUNK
N_CHUNKS = N_EDGES // CHUNK
HE = N_EDGES // 2
NCH = HE // CHUNK
NGH = HE // GRP
N_GRPS = N_EDGES // GRP

NC, NS = 2, 16
NW = NC * NS
CHUNKS_PER_W_H = -(-NCH // NW)
ROWS_PER_TILE = NP // NS


def _sc_body(ev0_hbm, ev1_hbm, ev2_hbm, idx_hbm, zeros_hbm,
             outp_hbm, outm_hbm,
             ev_v, p0_v, p1_v, p2_v, idx_v, acc_p, acc_m,
             sem_in, sem_idx, sem_sc):
    c = lax.axis_index("c")
    s = lax.axis_index("s")
    w = s * NC + c

    r0 = pl.multiple_of(s * ROWS_PER_TILE, 8)
    pltpu.sync_copy(zeros_hbm.at[pl.ds(r0, ROWS_PER_TILE), :],
                    acc_p.at[pl.ds(r0, ROWS_PER_TILE), :])
    pltpu.sync_copy(zeros_hbm.at[pl.ds(r0, ROWS_PER_TILE), :],
                    acc_m.at[pl.ds(r0, ROWS_PER_TILE), :])
    plsc.subcore_barrier()

    def stage_planes(cid):
        ebase = pl.multiple_of(cid * CHUNK, 8)
        for ev_hbm, pv in ((ev0_hbm, p0_v), (ev1_hbm, p1_v), (ev2_hbm, p2_v)):
            pltpu.async_copy(ev_hbm.at[pl.ds(ebase, CHUNK)], pv, sem_in)

    @pl.when(w < NCH)
    def _():
        stage_planes(w)

    def chunk_body(i, carry):
        cid = w + NW * i

        @pl.when(cid < NCH)
        def _():
            gbase = pl.multiple_of(cid * 2 * G_PER_CHUNK, 16)
            idx_dma = pltpu.async_copy(
                idx_hbm.at[pl.ds(gbase, 2 * G_PER_CHUNK), :], idx_v, sem_idx)
            for ref in (p0_v, p1_v, p2_v):
                pltpu.make_async_copy(ev0_hbm.at[pl.ds(0, CHUNK)], ref,
                                      sem_in).wait()

            lanes = lax.iota(jnp.int32, 16)
            cols3 = [jnp.full((16,), j, jnp.int32) for j in range(3)]

            def rep_body(k, carry3):
                rows = k * 16 + lanes
                for j, pv in enumerate((p0_v, p1_v, p2_v)):
                    v = pv[pl.ds(k * 16, 16)]
                    plsc.store_scatter(ev_v, [rows, cols3[j]], v)
                return carry3

            lax.fori_loop(0, CHUNK // 16, rep_body, 0, unroll=4)

            idx_dma.wait()
            scat = []
            for g in range(G_PER_CHUNK):
                src = ev_v.at[pl.ds(g * GRP, GRP), :]
                scat.append(pltpu.async_copy(src, acc_p.at[idx_v.at[2 * g]],
                                             sem_sc, add=True))
                scat.append(pltpu.async_copy(src, acc_m.at[idx_v.at[2 * g + 1]],
                                             sem_sc, add=True))

            cid2 = cid + NW

            @pl.when(cid2 < NCH)
            def _():
                stage_planes(cid2)

            for d in scat:
                d.wait()
        return carry

    lax.fori_loop(0, CHUNKS_PER_W_H, chunk_body, 0)

    plsc.subcore_barrier()
    pltpu.sync_copy(acc_p.at[pl.ds(r0, ROWS_PER_TILE), :],
                    outp_hbm.at[c, pl.ds(r0, ROWS_PER_TILE), :])
    pltpu.sync_copy(acc_m.at[pl.ds(r0, ROWS_PER_TILE), :],
                    outm_hbm.at[c, pl.ds(r0, ROWS_PER_TILE), :])


def _scatter_partials(ev0, ev1, ev2, idx_il, zeros):
    mesh = plsc.VectorSubcoreMesh(core_axis_name="c", subcore_axis_name="s")
    return pl.kernel(
        _sc_body,
        compiler_params=pltpu.CompilerParams(use_tc_tiling_on_sc=False,
                                             needs_layout_passes=False),
        out_type=(jax.ShapeDtypeStruct((NC, NP, D), jnp.float32),
                  jax.ShapeDtypeStruct((NC, NP, D), jnp.float32)),
        mesh=mesh,
        scratch_types=[
            pltpu.VMEM((CHUNK, D), jnp.float32),
            pltpu.VMEM((CHUNK,), jnp.float32),
            pltpu.VMEM((CHUNK,), jnp.float32),
            pltpu.VMEM((CHUNK,), jnp.float32),
            pltpu.VMEM((2 * G_PER_CHUNK, GRP), jnp.int32),
            pltpu.VMEM_SHARED((NP, D), jnp.float32),
            pltpu.VMEM_SHARED((NP, D), jnp.float32),
            pltpu.SemaphoreType.DMA,
            pltpu.SemaphoreType.DMA,
            pltpu.SemaphoreType.DMA,
        ],
    )(ev0, ev1, ev2, idx_il, zeros)


_CROWS = NP * D // 128


def _combine_body(c_ref, pa_ref, ma_ref, pb_ref, mb_ref, o_ref):
    t = ((pa_ref[0] + pa_ref[1] + pb_ref[0] + pb_ref[1])
         - (ma_ref[0] + ma_ref[1] + mb_ref[0] + mb_ref[1]))
    o_ref[...] = t * c_ref[...]


def _combine(pa, ma, pb, mb, cpat):
    args = [x.reshape(NC, _CROWS, 128) for x in (pa, ma, pb, mb)]
    return pl.pallas_call(
        _combine_body,
        out_shape=jax.ShapeDtypeStruct((_CROWS, 128), jnp.float32),
    )(cpat, *args)


def kernel(edge_vec, edge_idx, energy_coeff):
    zeros = jnp.zeros((NP, D), jnp.float32)
    halves = []
    for lo, hi in ((0, HE), (HE, N_EDGES)):
        idx_il = (edge_idx[:, lo:hi].reshape(2, NGH, GRP)
                  .transpose(1, 0, 2).reshape(2 * NGH, GRP))
        halves.append(_scatter_partials(
            edge_vec[lo:hi, 0], edge_vec[lo:hi, 1], edge_vec[lo:hi, 2],
            idx_il, zeros))
    (pa, ma), (pb, mb) = halves
    cpat = jnp.tile(jnp.pad(2.0 * energy_coeff, (0, D - 3)), 128 // D)
    combined = _combine(pa, ma, pb, mb, cpat.reshape(1, 128))
    return combined.reshape(NP, D)[:N_NODES, :3]


# speedup vs baseline: 1.1175x; 1.1175x over previous
import jax
import jax.numpy as jnp
from jax import lax
from jax.experimental import pallas as pl
from jax.experimental.pallas import tpu as pltpu
from jax.experimental.pallas import tpu_sc as plsc

N_NODES = 100000
N_EDGES = 6400000
NP = 100096
D = 8

GRP = 128
G_PER_CHUNK = 8
CHUNK = GRP * G_PER_CHUNK
N_CHUNKS = N_EDGES // CHUNK
N_GRPS = N_EDGES // GRP

NC, NS = 2, 16
NW = NC * NS
CHUNKS_PER_W = -(-N_CHUNKS // NW)
ROWS_PER_TILE = NP // NS


def _sc_body(ev0_hbm, ev1_hbm, ev2_hbm, idx_hbm, zeros_hbm,
             outp_hbm, outm_hbm,
             ev_v, p0_v, p1_v, p2_v, idx_v, acc_p, acc_m,
             sem_in, sem_idx, sem_sc):
    c = lax.axis_index("c")
    s = lax.axis_index("s")
    w = s * NC + c

    r0 = pl.multiple_of(s * ROWS_PER_TILE, 8)
    pltpu.sync_copy(zeros_hbm.at[pl.ds(r0, ROWS_PER_TILE), :],
                    acc_p.at[pl.ds(r0, ROWS_PER_TILE), :])
    pltpu.sync_copy(zeros_hbm.at[pl.ds(r0, ROWS_PER_TILE), :],
                    acc_m.at[pl.ds(r0, ROWS_PER_TILE), :])
    plsc.subcore_barrier()

    def stage_planes(cid):
        ebase = pl.multiple_of(cid * CHUNK, 8)
        for ev_hbm, pv in ((ev0_hbm, p0_v), (ev1_hbm, p1_v), (ev2_hbm, p2_v)):
            pltpu.async_copy(ev_hbm.at[pl.ds(ebase, CHUNK)], pv, sem_in)

    @pl.when(w < N_CHUNKS)
    def _():
        stage_planes(w)

    def chunk_body(i, carry):
        cid = w + NW * i

        @pl.when(cid < N_CHUNKS)
        def _():
            gbase = pl.multiple_of(cid * 2 * G_PER_CHUNK, 16)
            idx_dma = pltpu.async_copy(
                idx_hbm.at[pl.ds(gbase, 2 * G_PER_CHUNK), :], idx_v, sem_idx)
            for ref in (p0_v, p1_v, p2_v):
                pltpu.make_async_copy(ev0_hbm.at[pl.ds(0, CHUNK)], ref,
                                      sem_in).wait()

            lanes = lax.iota(jnp.int32, 16)
            cols3 = [jnp.full((16,), j, jnp.int32) for j in range(3)]

            def rep_body(k, carry3):
                rows = k * 16 + lanes
                for j, pv in enumerate((p0_v, p1_v, p2_v)):
                    v = pv[pl.ds(k * 16, 16)]
                    plsc.store_scatter(ev_v, [rows, cols3[j]], v)
                return carry3

            lax.fori_loop(0, CHUNK // 16, rep_body, 0, unroll=4)

            idx_dma.wait()
            scat = []
            for g in range(G_PER_CHUNK):
                src = ev_v.at[pl.ds(g * GRP, GRP), :]
                scat.append(pltpu.async_copy(src, acc_p.at[idx_v.at[2 * g]],
                                             sem_sc, add=True))
                scat.append(pltpu.async_copy(src, acc_m.at[idx_v.at[2 * g + 1]],
                                             sem_sc, add=True))

            cid2 = cid + NW

            @pl.when(cid2 < N_CHUNKS)
            def _():
                stage_planes(cid2)

            for d in scat:
                d.wait()
        return carry

    lax.fori_loop(0, CHUNKS_PER_W, chunk_body, 0)

    plsc.subcore_barrier()
    pltpu.sync_copy(acc_p.at[pl.ds(r0, ROWS_PER_TILE), :],
                    outp_hbm.at[c, pl.ds(r0, ROWS_PER_TILE), :])
    pltpu.sync_copy(acc_m.at[pl.ds(r0, ROWS_PER_TILE), :],
                    outm_hbm.at[c, pl.ds(r0, ROWS_PER_TILE), :])


def _scatter_partials(ev0, ev1, ev2, idx_il, zeros):
    mesh = plsc.VectorSubcoreMesh(core_axis_name="c", subcore_axis_name="s")
    return pl.kernel(
        _sc_body,
        compiler_params=pltpu.CompilerParams(use_tc_tiling_on_sc=False,
                                             needs_layout_passes=False),
        out_type=(jax.ShapeDtypeStruct((NC, NP, D), jnp.float32),
                  jax.ShapeDtypeStruct((NC, NP, D), jnp.float32)),
        mesh=mesh,
        scratch_types=[
            pltpu.VMEM((CHUNK, D), jnp.float32),
            pltpu.VMEM((CHUNK,), jnp.float32),
            pltpu.VMEM((CHUNK,), jnp.float32),
            pltpu.VMEM((CHUNK,), jnp.float32),
            pltpu.VMEM((2 * G_PER_CHUNK, GRP), jnp.int32),
            pltpu.VMEM_SHARED((NP, D), jnp.float32),
            pltpu.VMEM_SHARED((NP, D), jnp.float32),
            pltpu.SemaphoreType.DMA,
            pltpu.SemaphoreType.DMA,
            pltpu.SemaphoreType.DMA,
        ],
    )(ev0, ev1, ev2, idx_il, zeros)


_CROWS = NP * D // 128


def _combine_body(c_ref, p_ref, m_ref, o_ref):
    t = (p_ref[0] + p_ref[1]) - (m_ref[0] + m_ref[1])
    o_ref[...] = t * c_ref[...]


def _combine(outp, outm, cpat):
    p = outp.reshape(NC, _CROWS, 128)
    m = outm.reshape(NC, _CROWS, 128)
    return pl.pallas_call(
        _combine_body,
        out_shape=jax.ShapeDtypeStruct((_CROWS, 128), jnp.float32),
    )(cpat, p, m)


def kernel(edge_vec, edge_idx, energy_coeff):
    idx_il = (edge_idx.reshape(2, N_GRPS, GRP)
              .transpose(1, 0, 2).reshape(2 * N_GRPS, GRP))
    zeros = jnp.zeros((NP, D), jnp.float32)
    outp, outm = _scatter_partials(edge_vec[:, 0], edge_vec[:, 1],
                                   edge_vec[:, 2], idx_il, zeros)
    cpat = jnp.tile(jnp.pad(2.0 * energy_coeff, (0, D - 3)), 128 // D)
    combined = _combine(outp, outm, cpat.reshape(1, 128))
    return combined.reshape(NP, D)[:N_NODES, :3]
